# baseline (speedup 1.0000x reference)
import functools

import jax
import jax.numpy as jnp
from jax import lax
from jax.experimental import pallas as pl
from jax.experimental.pallas import tpu as pltpu
from jax.experimental.pallas import tpu_sc as plsc

_N = 10000
_E = 320000
_D = 128
_H = 128
_C = 64

_NC = 2
_NS = 16
_NW = _NC * _NS

_N_PAD = 10240
_ROWS_PER_TILE = _N_PAD // _NS
_CHUNK = 128
_NCHUNK = 80
_TILE_E = _NCHUNK * _CHUNK
_E_PAD = _TILE_E * _NW
_ECHUNKS = _E_PAD // _CHUNK
_GROWS = _N_PAD // _NW

_BLK = 512
_GRID = _N_PAD // _BLK

_mesh = plsc.VectorSubcoreMesh(core_axis_name="c", subcore_axis_name="s")


def _sc_agg_body(table, src, dst, zrows, out, acc, sidx0, sidx1, didx0, didx1,
                 rows0, rows1, semg0, semg1, semi0, semi1):
    c = lax.axis_index("c")
    s = lax.axis_index("s")
    wid = c * _NS + s
    rs = s * _ROWS_PER_TILE
    ebase = wid * _TILE_E
    pltpu.sync_copy(zrows.at[pl.ds(rs, _ROWS_PER_TILE)],
                    acc.at[pl.ds(rs, _ROWS_PER_TILE)])
    plsc.subcore_barrier()

    def off(b):
        return ebase + lax.rem(b, _NCHUNK) * _CHUNK

    pltpu.sync_copy(src.at[pl.ds(ebase, _CHUNK)], sidx0)
    pltpu.sync_copy(dst.at[pl.ds(ebase, _CHUNK)], didx0)
    pltpu.async_copy(table.at[sidx0], rows0, semg0)
    pltpu.async_copy(src.at[pl.ds(off(1), _CHUNK)], sidx1, semi1)
    pltpu.async_copy(dst.at[pl.ds(off(1), _CHUNK)], didx1, semi1)

    def body(j, carry):
        b1 = 2 * j + 1
        o1, o2, o3 = off(b1), off(b1 + 1), off(b1 + 2)
        pltpu.make_async_copy(src.at[pl.ds(o1, _CHUNK)], sidx1, semi1).wait()
        pltpu.make_async_copy(dst.at[pl.ds(o1, _CHUNK)], didx1, semi1).wait()
        pltpu.async_copy(table.at[sidx1], rows1, semg1)
        pltpu.make_async_copy(table.at[sidx0], rows0, semg0).wait()
        pltpu.sync_copy(rows0, acc.at[didx0], add=True)
        pltpu.async_copy(src.at[pl.ds(o2, _CHUNK)], sidx0, semi0)
        pltpu.async_copy(dst.at[pl.ds(o2, _CHUNK)], didx0, semi0)
        pltpu.make_async_copy(src.at[pl.ds(o2, _CHUNK)], sidx0, semi0).wait()
        pltpu.make_async_copy(dst.at[pl.ds(o2, _CHUNK)], didx0, semi0).wait()
        pltpu.async_copy(table.at[sidx0], rows0, semg0)
        pltpu.make_async_copy(table.at[sidx1], rows1, semg1).wait()
        pltpu.sync_copy(rows1, acc.at[didx1], add=True)
        pltpu.async_copy(src.at[pl.ds(o3, _CHUNK)], sidx1, semi1)
        pltpu.async_copy(dst.at[pl.ds(o3, _CHUNK)], didx1, semi1)
        return carry

    lax.fori_loop(0, _NCHUNK // 2, body, 0)
    pltpu.make_async_copy(src.at[pl.ds(ebase, _CHUNK)], sidx1, semi1).wait()
    pltpu.make_async_copy(dst.at[pl.ds(ebase, _CHUNK)], didx1, semi1).wait()
    pltpu.make_async_copy(table.at[sidx0], rows0, semg0).wait()
    plsc.subcore_barrier()
    pltpu.sync_copy(acc.at[pl.ds(rs, _ROWS_PER_TILE)],
                    out.at[pl.ds(c * _N_PAD + rs, _ROWS_PER_TILE)])


_sc_agg = pl.kernel(
    _sc_agg_body,
    out_type=jax.ShapeDtypeStruct((_NC * _N_PAD, _D), jnp.float32),
    mesh=_mesh,
    scratch_types=[
        pltpu.VMEM_SHARED((_N_PAD, _D), jnp.float32),
        pltpu.VMEM((_CHUNK,), jnp.int32),
        pltpu.VMEM((_CHUNK,), jnp.int32),
        pltpu.VMEM((_CHUNK,), jnp.int32),
        pltpu.VMEM((_CHUNK,), jnp.int32),
        pltpu.VMEM((_CHUNK, _D), jnp.float32),
        pltpu.VMEM((_CHUNK, _D), jnp.float32),
        pltpu.SemaphoreType.DMA,
        pltpu.SemaphoreType.DMA,
        pltpu.SemaphoreType.DMA,
        pltpu.SemaphoreType.DMA,
    ],
)


_GCHUNK = 64
_NGCHUNK = _GROWS // _GCHUNK


def _sc_prep_body(idp, srcb, dstb, dsta, x0, zcnt, ones,
                  idc_out, x0b_out, cnt_out, cntb_out,
                  cnt_acc, cntb_acc, id_v, idx_a, idx_b, idx_c,
                  idc_buf, ones_v, grows, sem_a, sem_b, sem_g, sem_l):
    c = lax.axis_index("c")
    s = lax.axis_index("s")
    wid = c * _NS + s
    rs = s * _ROWS_PER_TILE
    pltpu.sync_copy(zcnt.at[pl.ds(rs, _ROWS_PER_TILE)],
                    cnt_acc.at[pl.ds(rs, _ROWS_PER_TILE)])
    pltpu.sync_copy(zcnt.at[pl.ds(rs, _ROWS_PER_TILE)],
                    cntb_acc.at[pl.ds(rs, _ROWS_PER_TILE)])
    pltpu.sync_copy(ones, ones_v)
    pltpu.sync_copy(idp, id_v)
    plsc.subcore_barrier()
    ebase = wid * _TILE_E

    def cbody(i, carry):
        base = ebase + i * _CHUNK
        la = pltpu.async_copy(dsta.at[pl.ds(base, _CHUNK)], idx_a, sem_l)
        lb = pltpu.async_copy(dstb.at[pl.ds(base, _CHUNK)], idx_b, sem_l)
        lc = pltpu.async_copy(srcb.at[pl.ds(base, _CHUNK)], idx_c, sem_l)
        la.wait()
        lb.wait()
        lc.wait()
        ca = pltpu.async_copy(ones_v, cnt_acc.at[idx_a], sem_a, add=True)
        cb = pltpu.async_copy(ones_v, cntb_acc.at[idx_b], sem_b, add=True)
        cg = pltpu.async_copy(idp.at[idx_c],
                              idc_buf.at[pl.ds(i * _CHUNK, _CHUNK)], sem_g)
        ca.wait()
        cb.wait()
        cg.wait()
        return carry

    lax.fori_loop(0, _NCHUNK, cbody, 0)
    pltpu.sync_copy(idc_buf, idc_out.at[pl.ds(ebase, _TILE_E)])

    rbase = wid * _GROWS

    def gchunk(j, carry):
        b = rbase + j * _GCHUNK
        pltpu.async_copy(x0.at[id_v.at[pl.ds(b, _GCHUNK)]], grows,
                         sem_g).wait()
        pltpu.sync_copy(grows, x0b_out.at[pl.ds(b, _GCHUNK)])
        return carry

    lax.fori_loop(0, _NGCHUNK, gchunk, 0)

    plsc.subcore_barrier()
    pltpu.sync_copy(cnt_acc.at[pl.ds(rs, _ROWS_PER_TILE)],
                    cnt_out.at[pl.ds(c * _N_PAD + rs, _ROWS_PER_TILE)])
    pltpu.sync_copy(cntb_acc.at[pl.ds(rs, _ROWS_PER_TILE)],
                    cntb_out.at[pl.ds(c * _N_PAD + rs, _ROWS_PER_TILE)])


_sc_prep = pl.kernel(
    _sc_prep_body,
    out_type=(
        jax.ShapeDtypeStruct((_E_PAD,), jnp.int32),
        jax.ShapeDtypeStruct((_N_PAD, _D), jnp.float32),
        jax.ShapeDtypeStruct((_NC * _N_PAD,), jnp.float32),
        jax.ShapeDtypeStruct((_NC * _N_PAD,), jnp.float32),
    ),
    mesh=_mesh,
    scratch_types=[
        pltpu.VMEM_SHARED((_N_PAD,), jnp.float32),
        pltpu.VMEM_SHARED((_N_PAD,), jnp.float32),
        pltpu.VMEM((_N_PAD,), jnp.int32),
        pltpu.VMEM((_CHUNK,), jnp.int32),
        pltpu.VMEM((_CHUNK,), jnp.int32),
        pltpu.VMEM((_CHUNK,), jnp.int32),
        pltpu.VMEM((_TILE_E,), jnp.int32),
        pltpu.VMEM((_CHUNK,), jnp.float32),
        pltpu.VMEM((_GCHUNK, _D), jnp.float32),
        pltpu.SemaphoreType.DMA,
        pltpu.SemaphoreType.DMA,
        pltpu.SemaphoreType.DMA,
        pltpu.SemaphoreType.DMA,
    ],
)


def _mm(a, b):
    return jnp.dot(a, b, preferred_element_type=jnp.float32)


def _tc1_body(x0, x0b, s0a, s0b, sb0a, sb0b, cna, cnb, cba, cbb,
              w0lt, b0l, w0rt, r_ref, x1_out, xm2_out):
    r = r_ref[0, 0]
    inv = 1.0 / jnp.maximum(cna[...] + cnb[...], 1.0)
    invb = 1.0 / jnp.maximum(cba[...] + cbb[...], 1.0)
    mean0 = (s0a[...] + s0b[...]) * inv
    a0 = _mm(mean0, w0lt[...]) + b0l[...]
    x0v = x0[...]
    x1 = jnp.maximum(a0 + _mm(x0v, w0rt[...]), 0.0)
    xmix = r * x0v + (1.0 - r) * x0b[...]
    t = _mm(xmix, w0rt[...])
    xnew = jnp.maximum(a0 + t, 0.0)
    meanb = (sb0a[...] + sb0b[...]) * invb
    xnewb = jnp.maximum(_mm(meanb, w0lt[...]) + b0l[...] + t, 0.0)
    x1_out[...] = x1
    xm2_out[...] = r * xnew + (1.0 - r) * xnewb


def _tc2_body(xm2, s1a, s1b, sb1a, sb1b, cna, cnb, cba, cbb,
              w1lt, b1l, w1rt, wlint, blin, r_ref, out):
    r = r_ref[0, 0]
    inv = 1.0 / jnp.maximum(cna[...] + cnb[...], 1.0)
    invb = 1.0 / jnp.maximum(cba[...] + cbb[...], 1.0)
    mean1 = (s1a[...] + s1b[...]) * inv
    meanb1 = (sb1a[...] + sb1b[...]) * invb
    xm2v = xm2[...]
    t2 = _mm(xm2v, w1rt[...])
    xnew = jnp.maximum(_mm(mean1, w1lt[...]) + b1l[...] + t2, 0.0)
    xnewb = jnp.maximum(_mm(meanb1, w1lt[...]) + b1l[...] + t2, 0.0)
    xm3 = r * xnew + (1.0 - r) * xnewb
    logits = _mm(xm3, wlint[...]) + blin[...]
    m = jnp.max(logits, axis=-1, keepdims=True)
    lse = jnp.log(jnp.sum(jnp.exp(logits - m), axis=-1, keepdims=True)) + m
    out[...] = logits - lse


def _row_spec(width):
    return pl.BlockSpec((_BLK, width), lambda i: (i, 0))


def _full_spec(shape):
    return pl.BlockSpec(shape, lambda i: tuple(0 for _ in shape))


_tc1 = pl.pallas_call(
    _tc1_body,
    grid=(_GRID,),
    in_specs=[
        _row_spec(_D), _row_spec(_D),
        _row_spec(_D), _row_spec(_D),
        _row_spec(_D), _row_spec(_D),
        _row_spec(1), _row_spec(1), _row_spec(1), _row_spec(1),
        _full_spec((_D, _H)), _full_spec((1, _H)), _full_spec((_D, _H)),
        _full_spec((1, 1)),
    ],
    out_specs=[_row_spec(_H), _row_spec(_H)],
    out_shape=[
        jax.ShapeDtypeStruct((_N_PAD, _H), jnp.float32),
        jax.ShapeDtypeStruct((_N_PAD, _H), jnp.float32),
    ],
)

_tc2 = pl.pallas_call(
    _tc2_body,
    grid=(_GRID,),
    in_specs=[
        _row_spec(_H),
        _row_spec(_H), _row_spec(_H),
        _row_spec(_H), _row_spec(_H),
        _row_spec(1), _row_spec(1), _row_spec(1), _row_spec(1),
        _full_spec((_H, _H)), _full_spec((1, _H)), _full_spec((_H, _H)),
        _full_spec((_H, _C)), _full_spec((1, _C)),
        _full_spec((1, 1)),
    ],
    out_specs=_row_spec(_C),
    out_shape=jax.ShapeDtypeStruct((_N_PAD, _C), jnp.float32),
)


def kernel(x0, adj, adj_b, mix_ratio, id_old_value_new, W0l, b0l, W0r,
           W1l, b1l, W1r, Wlin, blin):
    pe = _E_PAD - _E
    pn = _N_PAD - _N
    x0p = jnp.pad(x0, ((0, pn), (0, 0)))
    zeroe = jnp.zeros((pe,), jnp.int32)
    dummye = jnp.full((pe,), _N, jnp.int32)
    src_a = jnp.concatenate([adj[0], zeroe])
    dst_a = jnp.concatenate([adj[1], dummye])
    src_b = jnp.concatenate([adj_b[0], zeroe])
    dst_b = jnp.concatenate([adj_b[1], dummye])
    idp = jnp.concatenate([id_old_value_new, jnp.zeros((pn,), jnp.int32)])
    zrows = jnp.zeros((_N_PAD, _D), jnp.float32)
    zcnt = jnp.zeros((_N_PAD,), jnp.float32)
    ones = jnp.ones((_CHUNK,), jnp.float32)
    r = jnp.reshape(mix_ratio, (1, 1)).astype(jnp.float32)
    w0lt, w0rt = W0l.T, W0r.T
    w1lt, w1rt = W1l.T, W1r.T
    wlint = Wlin.T
    b0l2 = jnp.reshape(b0l, (1, _H))
    b1l2 = jnp.reshape(b1l, (1, _H))
    blin2 = jnp.reshape(blin, (1, _C))

    idc, x0b, cntp, cntbp = _sc_prep(idp, src_b, dst_b, dst_a, x0p, zcnt, ones)
    s0 = _sc_agg(x0p, src_a, dst_a, zrows)
    sb0 = _sc_agg(x0p, idc, dst_b, zrows)

    cna = cntp[:_N_PAD].reshape(_N_PAD, 1)
    cnb = cntp[_N_PAD:].reshape(_N_PAD, 1)
    cba = cntbp[:_N_PAD].reshape(_N_PAD, 1)
    cbb = cntbp[_N_PAD:].reshape(_N_PAD, 1)

    x1, xm2 = _tc1(x0p, x0b, s0[:_N_PAD], s0[_N_PAD:], sb0[:_N_PAD],
                   sb0[_N_PAD:], cna, cnb, cba, cbb, w0lt, b0l2, w0rt, r)

    s1 = _sc_agg(x1, src_a, dst_a, zrows)
    sb1 = _sc_agg(x1, idc, dst_b, zrows)

    out = _tc2(xm2, s1[:_N_PAD], s1[_N_PAD:], sb1[:_N_PAD], sb1[_N_PAD:],
               cna, cnb, cba, cbb, w1lt, b1l2, w1rt, wlint, blin2, r)
    return out[:_N]


# speedup vs baseline: 1.0318x; 1.0318x over previous
import functools

import jax
import jax.numpy as jnp
from jax import lax
from jax.experimental import pallas as pl
from jax.experimental.pallas import tpu as pltpu
from jax.experimental.pallas import tpu_sc as plsc

_N = 10000
_E = 320000
_D = 128
_H = 128
_C = 64

_NC = 2
_NS = 16
_NW = _NC * _NS

_N_PAD = 10240
_ROWS_PER_TILE = _N_PAD // _NS
_CHUNK = 128
_NCHUNK = 80
_TILE_E = _NCHUNK * _CHUNK
_E_PAD = _TILE_E * _NW
_ECHUNKS = _E_PAD // _CHUNK
_GROWS = _N_PAD // _NW

_BLK = 512
_GRID = _N_PAD // _BLK

_mesh = plsc.VectorSubcoreMesh(core_axis_name="c", subcore_axis_name="s")


def _sc_agg_body(table, packed, zrows, out, acc, pbuf, sidx0, sidx1,
                 didx0, didx1, rows0, rows1, sem0, sem1):
    c = lax.axis_index("c")
    s = lax.axis_index("s")
    wid = c * _NS + s
    rs = s * _ROWS_PER_TILE
    pltpu.sync_copy(zrows.at[pl.ds(rs, _ROWS_PER_TILE)],
                    acc.at[pl.ds(rs, _ROWS_PER_TILE)])
    pltpu.sync_copy(packed.at[pl.ds(wid * _NCHUNK, _NCHUNK)], pbuf)
    plsc.subcore_barrier()

    def unpack(i, sidx, didx):
        for k in range(_CHUNK // 16):
            sl = pl.ds(k * 16, 16)
            v = pbuf[i, sl]
            sidx[sl] = jnp.bitwise_and(v, 0xFFFF)
            didx[sl] = lax.shift_right_logical(v, 16)

    unpack(0, sidx0, didx0)
    pltpu.async_copy(table.at[sidx0], rows0, sem0)

    def body(j, carry):
        b1 = 2 * j + 1
        b2 = lax.rem(b1 + 1, _NCHUNK)
        unpack(b1, sidx1, didx1)
        pltpu.async_copy(table.at[sidx1], rows1, sem1)
        pltpu.make_async_copy(table.at[sidx0], rows0, sem0).wait()
        pltpu.sync_copy(rows0, acc.at[didx0], add=True)
        unpack(b2, sidx0, didx0)
        pltpu.async_copy(table.at[sidx0], rows0, sem0)
        pltpu.make_async_copy(table.at[sidx1], rows1, sem1).wait()
        pltpu.sync_copy(rows1, acc.at[didx1], add=True)
        return carry

    lax.fori_loop(0, _NCHUNK // 2, body, 0)
    pltpu.make_async_copy(table.at[sidx0], rows0, sem0).wait()
    plsc.subcore_barrier()
    pltpu.sync_copy(acc.at[pl.ds(rs, _ROWS_PER_TILE)],
                    out.at[pl.ds(c * _N_PAD + rs, _ROWS_PER_TILE)])


_sc_agg = pl.kernel(
    _sc_agg_body,
    out_type=jax.ShapeDtypeStruct((_NC * _N_PAD, _D), jnp.float32),
    mesh=_mesh,
    scratch_types=[
        pltpu.VMEM_SHARED((_N_PAD, _D), jnp.float32),
        pltpu.VMEM((_NCHUNK, _CHUNK), jnp.int32),
        pltpu.VMEM((_CHUNK,), jnp.int32),
        pltpu.VMEM((_CHUNK,), jnp.int32),
        pltpu.VMEM((_CHUNK,), jnp.int32),
        pltpu.VMEM((_CHUNK,), jnp.int32),
        pltpu.VMEM((_CHUNK, _D), jnp.float32),
        pltpu.VMEM((_CHUNK, _D), jnp.float32),
        pltpu.SemaphoreType.DMA,
        pltpu.SemaphoreType.DMA,
    ],
)


_GCHUNK = 64
_NGCHUNK = _GROWS // _GCHUNK


def _sc_prep_body(idp, srcb2, packedab, x0, zcnt, ones,
                  idc_out, x0b_out, cnt_out, cntb_out,
                  cnt_acc, cntb_acc, id_v, sbuf, abuf, idx_a, idx_b,
                  idc_buf, ones_v, grows, sem_a, sem_b, sem_g):
    c = lax.axis_index("c")
    s = lax.axis_index("s")
    wid = c * _NS + s
    rs = s * _ROWS_PER_TILE
    pltpu.sync_copy(zcnt.at[pl.ds(rs, _ROWS_PER_TILE)],
                    cnt_acc.at[pl.ds(rs, _ROWS_PER_TILE)])
    pltpu.sync_copy(zcnt.at[pl.ds(rs, _ROWS_PER_TILE)],
                    cntb_acc.at[pl.ds(rs, _ROWS_PER_TILE)])
    pltpu.sync_copy(ones, ones_v)
    pltpu.sync_copy(idp, id_v)
    pltpu.sync_copy(srcb2.at[pl.ds(wid * _NCHUNK, _NCHUNK)], sbuf)
    pltpu.sync_copy(packedab.at[pl.ds(wid * _NCHUNK, _NCHUNK)], abuf)
    plsc.subcore_barrier()
    ebase = wid * _TILE_E

    def cbody(i, carry):
        for k in range(_CHUNK // 16):
            sl = pl.ds(k * 16, 16)
            v = abuf[i, sl]
            idx_a[sl] = jnp.bitwise_and(v, 0xFFFF)
            idx_b[sl] = lax.shift_right_logical(v, 16)
        ca = pltpu.async_copy(ones_v, cnt_acc.at[idx_a], sem_a, add=True)
        cb = pltpu.async_copy(ones_v, cntb_acc.at[idx_b], sem_b, add=True)
        cg = pltpu.async_copy(idp.at[sbuf.at[i]],
                              idc_buf.at[pl.ds(i * _CHUNK, _CHUNK)], sem_g)
        ca.wait()
        cb.wait()
        cg.wait()
        return carry

    lax.fori_loop(0, _NCHUNK, cbody, 0)
    pltpu.sync_copy(idc_buf, idc_out.at[pl.ds(ebase, _TILE_E)])

    rbase = wid * _GROWS

    def gchunk(j, carry):
        b = rbase + j * _GCHUNK
        pltpu.async_copy(x0.at[id_v.at[pl.ds(b, _GCHUNK)]], grows,
                         sem_g).wait()
        pltpu.sync_copy(grows, x0b_out.at[pl.ds(b, _GCHUNK)])
        return carry

    lax.fori_loop(0, _NGCHUNK, gchunk, 0)

    plsc.subcore_barrier()
    pltpu.sync_copy(cnt_acc.at[pl.ds(rs, _ROWS_PER_TILE)],
                    cnt_out.at[pl.ds(c * _N_PAD + rs, _ROWS_PER_TILE)])
    pltpu.sync_copy(cntb_acc.at[pl.ds(rs, _ROWS_PER_TILE)],
                    cntb_out.at[pl.ds(c * _N_PAD + rs, _ROWS_PER_TILE)])


_sc_prep = pl.kernel(
    _sc_prep_body,
    out_type=(
        jax.ShapeDtypeStruct((_E_PAD,), jnp.int32),
        jax.ShapeDtypeStruct((_N_PAD, _D), jnp.float32),
        jax.ShapeDtypeStruct((_NC * _N_PAD,), jnp.float32),
        jax.ShapeDtypeStruct((_NC * _N_PAD,), jnp.float32),
    ),
    mesh=_mesh,
    scratch_types=[
        pltpu.VMEM_SHARED((_N_PAD,), jnp.float32),
        pltpu.VMEM_SHARED((_N_PAD,), jnp.float32),
        pltpu.VMEM((_N_PAD,), jnp.int32),
        pltpu.VMEM((_NCHUNK, _CHUNK), jnp.int32),
        pltpu.VMEM((_NCHUNK, _CHUNK), jnp.int32),
        pltpu.VMEM((_CHUNK,), jnp.int32),
        pltpu.VMEM((_CHUNK,), jnp.int32),
        pltpu.VMEM((_TILE_E,), jnp.int32),
        pltpu.VMEM((_CHUNK,), jnp.float32),
        pltpu.VMEM((_GCHUNK, _D), jnp.float32),
        pltpu.SemaphoreType.DMA,
        pltpu.SemaphoreType.DMA,
        pltpu.SemaphoreType.DMA,
    ],
)


def _mm(a, b):
    return jnp.dot(a, b, preferred_element_type=jnp.float32)


def _tc1_body(x0, x0b, s0a, s0b, sb0a, sb0b, cna, cnb, cba, cbb,
              w0lt, b0l, w0rt, r_ref, x1_out, xm2_out):
    r = r_ref[0, 0]
    inv = 1.0 / jnp.maximum(cna[...] + cnb[...], 1.0)
    invb = 1.0 / jnp.maximum(cba[...] + cbb[...], 1.0)
    mean0 = (s0a[...] + s0b[...]) * inv
    a0 = _mm(mean0, w0lt[...]) + b0l[...]
    x0v = x0[...]
    x1 = jnp.maximum(a0 + _mm(x0v, w0rt[...]), 0.0)
    xmix = r * x0v + (1.0 - r) * x0b[...]
    t = _mm(xmix, w0rt[...])
    xnew = jnp.maximum(a0 + t, 0.0)
    meanb = (sb0a[...] + sb0b[...]) * invb
    xnewb = jnp.maximum(_mm(meanb, w0lt[...]) + b0l[...] + t, 0.0)
    x1_out[...] = x1
    xm2_out[...] = r * xnew + (1.0 - r) * xnewb


def _tc2_body(xm2, s1a, s1b, sb1a, sb1b, cna, cnb, cba, cbb,
              w1lt, b1l, w1rt, wlint, blin, r_ref, out):
    r = r_ref[0, 0]
    inv = 1.0 / jnp.maximum(cna[...] + cnb[...], 1.0)
    invb = 1.0 / jnp.maximum(cba[...] + cbb[...], 1.0)
    mean1 = (s1a[...] + s1b[...]) * inv
    meanb1 = (sb1a[...] + sb1b[...]) * invb
    xm2v = xm2[...]
    t2 = _mm(xm2v, w1rt[...])
    xnew = jnp.maximum(_mm(mean1, w1lt[...]) + b1l[...] + t2, 0.0)
    xnewb = jnp.maximum(_mm(meanb1, w1lt[...]) + b1l[...] + t2, 0.0)
    xm3 = r * xnew + (1.0 - r) * xnewb
    logits = _mm(xm3, wlint[...]) + blin[...]
    m = jnp.max(logits, axis=-1, keepdims=True)
    lse = jnp.log(jnp.sum(jnp.exp(logits - m), axis=-1, keepdims=True)) + m
    out[...] = logits - lse


def _row_spec(width):
    return pl.BlockSpec((_BLK, width), lambda i: (i, 0))


def _full_spec(shape):
    return pl.BlockSpec(shape, lambda i: tuple(0 for _ in shape))


_tc1 = pl.pallas_call(
    _tc1_body,
    grid=(_GRID,),
    in_specs=[
        _row_spec(_D), _row_spec(_D),
        _row_spec(_D), _row_spec(_D),
        _row_spec(_D), _row_spec(_D),
        _row_spec(1), _row_spec(1), _row_spec(1), _row_spec(1),
        _full_spec((_D, _H)), _full_spec((1, _H)), _full_spec((_D, _H)),
        _full_spec((1, 1)),
    ],
    out_specs=[_row_spec(_H), _row_spec(_H)],
    out_shape=[
        jax.ShapeDtypeStruct((_N_PAD, _H), jnp.float32),
        jax.ShapeDtypeStruct((_N_PAD, _H), jnp.float32),
    ],
)

_tc2 = pl.pallas_call(
    _tc2_body,
    grid=(_GRID,),
    in_specs=[
        _row_spec(_H),
        _row_spec(_H), _row_spec(_H),
        _row_spec(_H), _row_spec(_H),
        _row_spec(1), _row_spec(1), _row_spec(1), _row_spec(1),
        _full_spec((_H, _H)), _full_spec((1, _H)), _full_spec((_H, _H)),
        _full_spec((_H, _C)), _full_spec((1, _C)),
        _full_spec((1, 1)),
    ],
    out_specs=_row_spec(_C),
    out_shape=jax.ShapeDtypeStruct((_N_PAD, _C), jnp.float32),
)


def kernel(x0, adj, adj_b, mix_ratio, id_old_value_new, W0l, b0l, W0r,
           W1l, b1l, W1r, Wlin, blin):
    pe = _E_PAD - _E
    pn = _N_PAD - _N
    x0p = jnp.pad(x0, ((0, pn), (0, 0)))
    zeroe = jnp.zeros((pe,), jnp.int32)
    dummye = jnp.full((pe,), _N, jnp.int32)
    src_a = jnp.concatenate([adj[0], zeroe])
    dst_a = jnp.concatenate([adj[1], dummye])
    src_b = jnp.concatenate([adj_b[0], zeroe]).reshape(_ECHUNKS, _CHUNK)
    dst_b = jnp.concatenate([adj_b[1], dummye])
    pk_a = (src_a | (dst_a << 16)).reshape(_ECHUNKS, _CHUNK)
    pk_ab = (dst_a | (dst_b << 16)).reshape(_ECHUNKS, _CHUNK)
    idp = jnp.concatenate([id_old_value_new, jnp.zeros((pn,), jnp.int32)])
    zrows = jnp.zeros((_N_PAD, _D), jnp.float32)
    zcnt = jnp.zeros((_N_PAD,), jnp.float32)
    ones = jnp.ones((_CHUNK,), jnp.float32)
    r = jnp.reshape(mix_ratio, (1, 1)).astype(jnp.float32)
    w0lt, w0rt = W0l.T, W0r.T
    w1lt, w1rt = W1l.T, W1r.T
    wlint = Wlin.T
    b0l2 = jnp.reshape(b0l, (1, _H))
    b1l2 = jnp.reshape(b1l, (1, _H))
    blin2 = jnp.reshape(blin, (1, _C))

    idc, x0b, cntp, cntbp = _sc_prep(idp, src_b, pk_ab, x0p, zcnt, ones)
    dst_b2 = dst_b.reshape(_ECHUNKS, _CHUNK)
    pk_b = idc.reshape(_ECHUNKS, _CHUNK) | (dst_b2 << 16)
    s0 = _sc_agg(x0p, pk_a, zrows)
    sb0 = _sc_agg(x0p, pk_b, zrows)

    cna = cntp[:_N_PAD].reshape(_N_PAD, 1)
    cnb = cntp[_N_PAD:].reshape(_N_PAD, 1)
    cba = cntbp[:_N_PAD].reshape(_N_PAD, 1)
    cbb = cntbp[_N_PAD:].reshape(_N_PAD, 1)

    x1, xm2 = _tc1(x0p, x0b, s0[:_N_PAD], s0[_N_PAD:], sb0[:_N_PAD],
                   sb0[_N_PAD:], cna, cnb, cba, cbb, w0lt, b0l2, w0rt, r)

    s1 = _sc_agg(x1, pk_a, zrows)
    sb1 = _sc_agg(x1, pk_b, zrows)

    out = _tc2(xm2, s1[:_N_PAD], s1[_N_PAD:], sb1[:_N_PAD], sb1[_N_PAD:],
               cna, cnb, cba, cbb, w1lt, b1l2, w1rt, wlint, blin2, r)
    return out[:_N]


# speedup vs baseline: 1.4614x; 1.4163x over previous
import functools

import jax
import jax.numpy as jnp
from jax import lax
from jax.experimental import pallas as pl
from jax.experimental.pallas import tpu as pltpu
from jax.experimental.pallas import tpu_sc as plsc

_N = 10000
_E = 320000
_D = 128
_H = 128
_C = 64

_NC = 2
_NS = 16
_NW = _NC * _NS

_N_PAD = 10240
_ROWS_PER_TILE = _N_PAD // _NS
_CHUNK = 128
_NCHUNK = 80
_TILE_E = _NCHUNK * _CHUNK
_E_PAD = _TILE_E * _NW
_ECHUNKS = _E_PAD // _CHUNK
_GROWS = _N_PAD // _NW

_BLK = 512
_GRID = _N_PAD // _BLK


_PERM = tuple(
    32 * (j // 32) + ((j % 32) // 2) + (16 if j % 2 else 0)
    for j in range(_D)
)

_mesh = plsc.VectorSubcoreMesh(core_axis_name="c", subcore_axis_name="s")


def _sc_agg_body(table, packed, zrows, out, acc, pbuf, sidx0, sidx1,
                 didx0, didx1, ibuf0, ibuf1, rowsf, sem0, sem1):
    c = lax.axis_index("c")
    s = lax.axis_index("s")
    wid = c * _NS + s
    rs = s * _ROWS_PER_TILE
    pltpu.sync_copy(zrows.at[pl.ds(rs, _ROWS_PER_TILE)],
                    acc.at[pl.ds(rs, _ROWS_PER_TILE)])
    pltpu.sync_copy(packed.at[pl.ds(wid * _NCHUNK, _NCHUNK)], pbuf)
    plsc.subcore_barrier()

    def unpack_idx(i, sidx, didx):
        for k in range(_CHUNK // 16):
            sl = pl.ds(k * 16, 16)
            v = pbuf[i, sl]
            sidx[sl] = jnp.bitwise_and(v, 0xFFFF)
            didx[sl] = lax.shift_right_logical(v, 16)

    def to_f32(ibuf):
        def row(r, carry):
            for g in range(_D // 32):
                v = ibuf[r, pl.ds(g * 16, 16)]
                lo = lax.bitcast_convert_type(
                    lax.shift_left(v, 16), jnp.float32)
                hi = lax.bitcast_convert_type(
                    jnp.bitwise_and(v, jnp.int32(-65536)), jnp.float32)
                rowsf[r, pl.ds(g * 32, 16)] = lo
                rowsf[r, pl.ds(g * 32 + 16, 16)] = hi
            return carry
        lax.fori_loop(0, _CHUNK, row, 0)

    unpack_idx(0, sidx0, didx0)
    pltpu.async_copy(table.at[sidx0], ibuf0, sem0)

    def body(j, carry):
        b1 = 2 * j + 1
        b2 = lax.rem(b1 + 1, _NCHUNK)
        unpack_idx(b1, sidx1, didx1)
        pltpu.async_copy(table.at[sidx1], ibuf1, sem1)
        pltpu.make_async_copy(table.at[sidx0], ibuf0, sem0).wait()
        to_f32(ibuf0)
        pltpu.sync_copy(rowsf, acc.at[didx0], add=True)
        unpack_idx(b2, sidx0, didx0)
        pltpu.async_copy(table.at[sidx0], ibuf0, sem0)
        pltpu.make_async_copy(table.at[sidx1], ibuf1, sem1).wait()
        to_f32(ibuf1)
        pltpu.sync_copy(rowsf, acc.at[didx1], add=True)
        return carry

    lax.fori_loop(0, _NCHUNK // 2, body, 0)
    pltpu.make_async_copy(table.at[sidx0], ibuf0, sem0).wait()
    plsc.subcore_barrier()
    pltpu.sync_copy(acc.at[pl.ds(rs, _ROWS_PER_TILE)],
                    out.at[pl.ds(c * _N_PAD + rs, _ROWS_PER_TILE)])


_sc_agg = pl.kernel(
    _sc_agg_body,
    out_type=jax.ShapeDtypeStruct((_NC * _N_PAD, _D), jnp.float32),
    mesh=_mesh,
    compiler_params=pltpu.CompilerParams(use_tc_tiling_on_sc=False),
    scratch_types=[
        pltpu.VMEM_SHARED((_N_PAD, _D), jnp.float32),
        pltpu.VMEM((_NCHUNK, _CHUNK), jnp.int32),
        pltpu.VMEM((_CHUNK,), jnp.int32),
        pltpu.VMEM((_CHUNK,), jnp.int32),
        pltpu.VMEM((_CHUNK,), jnp.int32),
        pltpu.VMEM((_CHUNK,), jnp.int32),
        pltpu.VMEM((_CHUNK, _D // 2), jnp.int32),
        pltpu.VMEM((_CHUNK, _D // 2), jnp.int32),
        pltpu.VMEM((_CHUNK, _D), jnp.float32),
        pltpu.SemaphoreType.DMA,
        pltpu.SemaphoreType.DMA,
    ],
)


_GCHUNK = 64
_NGCHUNK = _GROWS // _GCHUNK


def _sc_prep_body(idp, srcb2, packedab, x0, zcnt, ones,
                  idc_out, x0b_out, cnt_out, cntb_out,
                  cnt_acc, cntb_acc, id_v, sbuf, abuf, idx_a, idx_b,
                  idc_buf, ones_v, grows, sem_a, sem_b, sem_g):
    c = lax.axis_index("c")
    s = lax.axis_index("s")
    wid = c * _NS + s
    rs = s * _ROWS_PER_TILE
    pltpu.sync_copy(zcnt.at[pl.ds(rs, _ROWS_PER_TILE)],
                    cnt_acc.at[pl.ds(rs, _ROWS_PER_TILE)])
    pltpu.sync_copy(zcnt.at[pl.ds(rs, _ROWS_PER_TILE)],
                    cntb_acc.at[pl.ds(rs, _ROWS_PER_TILE)])
    pltpu.sync_copy(ones, ones_v)
    pltpu.sync_copy(idp, id_v)
    pltpu.sync_copy(srcb2.at[pl.ds(wid * _NCHUNK, _NCHUNK)], sbuf)
    pltpu.sync_copy(packedab.at[pl.ds(wid * _NCHUNK, _NCHUNK)], abuf)
    plsc.subcore_barrier()
    ebase = wid * _TILE_E

    def cbody(i, carry):
        for k in range(_CHUNK // 16):
            sl = pl.ds(k * 16, 16)
            v = abuf[i, sl]
            idx_a[sl] = jnp.bitwise_and(v, 0xFFFF)
            idx_b[sl] = lax.shift_right_logical(v, 16)
        ca = pltpu.async_copy(ones_v, cnt_acc.at[idx_a], sem_a, add=True)
        cb = pltpu.async_copy(ones_v, cntb_acc.at[idx_b], sem_b, add=True)
        cg = pltpu.async_copy(idp.at[sbuf.at[i]],
                              idc_buf.at[pl.ds(i * _CHUNK, _CHUNK)], sem_g)
        ca.wait()
        cb.wait()
        cg.wait()
        return carry

    lax.fori_loop(0, _NCHUNK, cbody, 0)
    pltpu.sync_copy(idc_buf, idc_out.at[pl.ds(ebase, _TILE_E)])

    rbase = wid * _GROWS

    def gchunk(j, carry):
        b = rbase + j * _GCHUNK
        pltpu.async_copy(x0.at[id_v.at[pl.ds(b, _GCHUNK)]], grows,
                         sem_g).wait()
        pltpu.sync_copy(grows, x0b_out.at[pl.ds(b, _GCHUNK)])
        return carry

    lax.fori_loop(0, _NGCHUNK, gchunk, 0)

    plsc.subcore_barrier()
    pltpu.sync_copy(cnt_acc.at[pl.ds(rs, _ROWS_PER_TILE)],
                    cnt_out.at[pl.ds(c * _N_PAD + rs, _ROWS_PER_TILE)])
    pltpu.sync_copy(cntb_acc.at[pl.ds(rs, _ROWS_PER_TILE)],
                    cntb_out.at[pl.ds(c * _N_PAD + rs, _ROWS_PER_TILE)])


_sc_prep = pl.kernel(
    _sc_prep_body,
    out_type=(
        jax.ShapeDtypeStruct((_E_PAD,), jnp.int32),
        jax.ShapeDtypeStruct((_N_PAD, _D), jnp.float32),
        jax.ShapeDtypeStruct((_NC * _N_PAD,), jnp.float32),
        jax.ShapeDtypeStruct((_NC * _N_PAD,), jnp.float32),
    ),
    mesh=_mesh,
    scratch_types=[
        pltpu.VMEM_SHARED((_N_PAD,), jnp.float32),
        pltpu.VMEM_SHARED((_N_PAD,), jnp.float32),
        pltpu.VMEM((_N_PAD,), jnp.int32),
        pltpu.VMEM((_NCHUNK, _CHUNK), jnp.int32),
        pltpu.VMEM((_NCHUNK, _CHUNK), jnp.int32),
        pltpu.VMEM((_CHUNK,), jnp.int32),
        pltpu.VMEM((_CHUNK,), jnp.int32),
        pltpu.VMEM((_TILE_E,), jnp.int32),
        pltpu.VMEM((_CHUNK,), jnp.float32),
        pltpu.VMEM((_GCHUNK, _D), jnp.float32),
        pltpu.SemaphoreType.DMA,
        pltpu.SemaphoreType.DMA,
        pltpu.SemaphoreType.DMA,
    ],
)


def _mm(a, b):
    return jnp.dot(a, b, preferred_element_type=jnp.float32)


def _tc1_body(x0, x0b, s0a, s0b, sb0a, sb0b, cna, cnb, cba, cbb,
              w0lt, b0l, w0rt, r_ref, x1_out, xm2_out):
    r = r_ref[0, 0]
    inv = 1.0 / jnp.maximum(cna[...] + cnb[...], 1.0)
    invb = 1.0 / jnp.maximum(cba[...] + cbb[...], 1.0)
    mean0 = (s0a[...] + s0b[...]) * inv
    a0 = _mm(mean0, w0lt[...]) + b0l[...]
    x0v = x0[...]
    x1 = jnp.maximum(a0 + _mm(x0v, w0rt[...]), 0.0)
    xmix = r * x0v + (1.0 - r) * x0b[...]
    t = _mm(xmix, w0rt[...])
    xnew = jnp.maximum(a0 + t, 0.0)
    meanb = (sb0a[...] + sb0b[...]) * invb
    xnewb = jnp.maximum(_mm(meanb, w0lt[...]) + b0l[...] + t, 0.0)
    x1_out[...] = x1
    xm2_out[...] = r * xnew + (1.0 - r) * xnewb


def _tc2_body(xm2, s1a, s1b, sb1a, sb1b, cna, cnb, cba, cbb,
              w1lt, b1l, w1rt, wlint, blin, r_ref, out):
    r = r_ref[0, 0]
    inv = 1.0 / jnp.maximum(cna[...] + cnb[...], 1.0)
    invb = 1.0 / jnp.maximum(cba[...] + cbb[...], 1.0)
    mean1 = (s1a[...] + s1b[...]) * inv
    meanb1 = (sb1a[...] + sb1b[...]) * invb
    xm2v = xm2[...]
    t2 = _mm(xm2v, w1rt[...])
    xnew = jnp.maximum(_mm(mean1, w1lt[...]) + b1l[...] + t2, 0.0)
    xnewb = jnp.maximum(_mm(meanb1, w1lt[...]) + b1l[...] + t2, 0.0)
    xm3 = r * xnew + (1.0 - r) * xnewb
    logits = _mm(xm3, wlint[...]) + blin[...]
    m = jnp.max(logits, axis=-1, keepdims=True)
    lse = jnp.log(jnp.sum(jnp.exp(logits - m), axis=-1, keepdims=True)) + m
    out[...] = logits - lse


def _row_spec(width):
    return pl.BlockSpec((_BLK, width), lambda i: (i, 0))


def _full_spec(shape):
    return pl.BlockSpec(shape, lambda i: tuple(0 for _ in shape))


_tc1 = pl.pallas_call(
    _tc1_body,
    grid=(_GRID,),
    in_specs=[
        _row_spec(_D), _row_spec(_D),
        _row_spec(_D), _row_spec(_D),
        _row_spec(_D), _row_spec(_D),
        _row_spec(1), _row_spec(1), _row_spec(1), _row_spec(1),
        _full_spec((_D, _H)), _full_spec((1, _H)), _full_spec((_D, _H)),
        _full_spec((1, 1)),
    ],
    out_specs=[_row_spec(_H), _row_spec(_H)],
    out_shape=[
        jax.ShapeDtypeStruct((_N_PAD, _H), jnp.float32),
        jax.ShapeDtypeStruct((_N_PAD, _H), jnp.float32),
    ],
)

_tc2 = pl.pallas_call(
    _tc2_body,
    grid=(_GRID,),
    in_specs=[
        _row_spec(_H),
        _row_spec(_H), _row_spec(_H),
        _row_spec(_H), _row_spec(_H),
        _row_spec(1), _row_spec(1), _row_spec(1), _row_spec(1),
        _full_spec((_H, _H)), _full_spec((1, _H)), _full_spec((_H, _H)),
        _full_spec((_H, _C)), _full_spec((1, _C)),
        _full_spec((1, 1)),
    ],
    out_specs=_row_spec(_C),
    out_shape=jax.ShapeDtypeStruct((_N_PAD, _C), jnp.float32),
)


def kernel(x0, adj, adj_b, mix_ratio, id_old_value_new, W0l, b0l, W0r,
           W1l, b1l, W1r, Wlin, blin):
    pe = _E_PAD - _E
    pn = _N_PAD - _N
    x0p = jnp.pad(x0, ((0, pn), (0, 0)))
    zeroe = jnp.zeros((pe,), jnp.int32)
    dummye = jnp.full((pe,), _N, jnp.int32)
    src_a = jnp.concatenate([adj[0], zeroe])
    dst_a = jnp.concatenate([adj[1], dummye])
    src_b = jnp.concatenate([adj_b[0], zeroe]).reshape(_ECHUNKS, _CHUNK)
    dst_b = jnp.concatenate([adj_b[1], dummye])
    pk_a = (src_a | (dst_a << 16)).reshape(_ECHUNKS, _CHUNK)
    pk_ab = (dst_a | (dst_b << 16)).reshape(_ECHUNKS, _CHUNK)
    x0bf = lax.bitcast_convert_type(
        x0p[:, _PERM].astype(jnp.bfloat16).reshape(_N_PAD, _D // 2, 2),
        jnp.int32)
    idp = jnp.concatenate([id_old_value_new, jnp.zeros((pn,), jnp.int32)])
    zrows = jnp.zeros((_N_PAD, _D), jnp.float32)
    zcnt = jnp.zeros((_N_PAD,), jnp.float32)
    ones = jnp.ones((_CHUNK,), jnp.float32)
    r = jnp.reshape(mix_ratio, (1, 1)).astype(jnp.float32)
    w0lt, w0rt = W0l.T, W0r.T
    w1lt, w1rt = W1l.T, W1r.T
    wlint = Wlin.T
    b0l2 = jnp.reshape(b0l, (1, _H))
    b1l2 = jnp.reshape(b1l, (1, _H))
    blin2 = jnp.reshape(blin, (1, _C))

    idc, x0b, cntp, cntbp = _sc_prep(idp, src_b, pk_ab, x0p, zcnt, ones)
    dst_b2 = dst_b.reshape(_ECHUNKS, _CHUNK)
    pk_b = idc.reshape(_ECHUNKS, _CHUNK) | (dst_b2 << 16)
    s0 = _sc_agg(x0bf, pk_a, zrows)
    sb0 = _sc_agg(x0bf, pk_b, zrows)

    cna = cntp[:_N_PAD].reshape(_N_PAD, 1)
    cnb = cntp[_N_PAD:].reshape(_N_PAD, 1)
    cba = cntbp[:_N_PAD].reshape(_N_PAD, 1)
    cbb = cntbp[_N_PAD:].reshape(_N_PAD, 1)

    x1, xm2 = _tc1(x0p, x0b, s0[:_N_PAD], s0[_N_PAD:], sb0[:_N_PAD],
                   sb0[_N_PAD:], cna, cnb, cba, cbb, w0lt, b0l2, w0rt, r)

    x1bf = lax.bitcast_convert_type(
        x1[:, _PERM].astype(jnp.bfloat16).reshape(_N_PAD, _D // 2, 2),
        jnp.int32)
    s1 = _sc_agg(x1bf, pk_a, zrows)
    sb1 = _sc_agg(x1bf, pk_b, zrows)

    out = _tc2(xm2, s1[:_N_PAD], s1[_N_PAD:], sb1[:_N_PAD], sb1[_N_PAD:],
               cna, cnb, cba, cbb, w1lt, b1l2, w1rt, wlint, blin2, r)
    return out[:_N]
